# Initial kernel scaffold; baseline (speedup 1.0000x reference)
import functools

import jax
import jax.numpy as jnp
from jax import lax
from jax.experimental import pallas as pl
from jax.experimental.pallas import tpu as pltpu
from jax.experimental.pallas import tpu_sc as plsc

N = 10000
E = 320000
D = 128
DH = 64
NC = 2
NS = 16
L = 16

C = 128
NCH = -(-E // (NS * C))
EPT = NCH * C
EPAD = NS * EPT

RPT = 640
NPAD = NS * RPT
RC = 128
NRC = RPT // RC

_F32 = jnp.float32


def _rsqrt16(v):
    half = jnp.full((L,), 0.5, _F32)
    three_half = jnp.full((L,), 1.5, _F32)
    one = jnp.full((L,), 1, jnp.int32)
    magic = jnp.full((L,), 0x5F3759DF, jnp.int32)
    i = plsc.bitcast(v, jnp.int32)
    i = magic - lax.shift_right_arithmetic(i, one)
    y = plsc.bitcast(i, _F32)
    for _ in range(3):
        y = y * (three_half - half * v * y * y)
    return y


def _scale_rows(nbuf, sref, base):
    def body(i, _):
        dv = plsc.load_gather(sref, [jnp.full((L,), base + i, jnp.int32)])
        for q in range(DH // L):
            sl = pl.ds(q * L, L)
            nbuf[i, sl] = nbuf[i, sl] * dv
        return 0
    lax.fori_loop(0, RC, body, 0)


def _fill(ref, n, val):
    vec = jnp.full((L,), val, ref.dtype)
    def body(k, _):
        ref[pl.ds(k * L, L)] = vec
        return 0
    lax.fori_loop(0, n // L, body, 0)


def _sc_body(xs, rowc, colc, out,
             y_sh, z_sh, deg_sh,
             col_v, row_v, gbuf, nbuf, degt, dtile, itile, ones_v):
    c = lax.axis_index("c")
    s = lax.axis_index("s")

    pltpu.sync_copy(colc.at[s], col_v)
    pltpu.sync_copy(rowc.at[s], row_v)
    _fill(ones_v, C, 1.0)
    _fill(degt, RPT, 1.0)
    pltpu.sync_copy(degt, deg_sh.at[pl.ds(s * RPT, RPT)])
    plsc.subcore_barrier()

    def hist(j, _):
        pltpu.sync_copy(ones_v, deg_sh.at[col_v.at[j]], add=True)
        return 0
    lax.fori_loop(0, NCH, hist, 0)
    plsc.subcore_barrier()

    pltpu.sync_copy(deg_sh.at[pl.ds(s * RPT, RPT)], degt)
    def dcalc(k, _):
        sl = pl.ds(k * L, L)
        y = _rsqrt16(degt[sl])
        dtile[sl] = y
        itile[sl] = y * y
        return 0
    lax.fori_loop(0, RPT // L, dcalc, 0)

    def prescale(r, _):
        g0 = s * RPT + r * RC
        pltpu.sync_copy(xs.at[c, pl.ds(g0, RC), :], nbuf)
        _scale_rows(nbuf, dtile, r * RC)
        pltpu.sync_copy(nbuf, y_sh.at[pl.ds(g0, RC)])
        pltpu.sync_copy(nbuf, z_sh.at[pl.ds(g0, RC)])
        return 0
    lax.fori_loop(0, NRC, prescale, 0)
    plsc.subcore_barrier()

    def edges(j, _):
        pltpu.sync_copy(y_sh.at[col_v.at[j]], gbuf)
        pltpu.sync_copy(gbuf, z_sh.at[row_v.at[j]], add=True)
        return 0
    lax.fori_loop(0, NCH, edges, 0)
    plsc.subcore_barrier()

    def midscale(r, _):
        g0 = s * RPT + r * RC
        pltpu.sync_copy(z_sh.at[pl.ds(g0, RC)], nbuf)
        _scale_rows(nbuf, itile, r * RC)
        pltpu.sync_copy(nbuf, y_sh.at[pl.ds(g0, RC)])
        pltpu.sync_copy(nbuf, z_sh.at[pl.ds(g0, RC)])
        return 0
    lax.fori_loop(0, NRC, midscale, 0)
    plsc.subcore_barrier()

    lax.fori_loop(0, NCH, edges, 0)
    plsc.subcore_barrier()

    def finalize(r, _):
        g0 = s * RPT + r * RC
        pltpu.sync_copy(z_sh.at[pl.ds(g0, RC)], nbuf)
        _scale_rows(nbuf, dtile, r * RC)
        pltpu.sync_copy(nbuf, out.at[c, pl.ds(g0, RC), :])
        return 0
    lax.fori_loop(0, NRC, finalize, 0)


def _mm_body(xa_ref, xb_ref, wa_ref, wb_ref, b_ref, o_ref):
    o_ref[...] = (
        jnp.dot(xa_ref[...], wa_ref[...], preferred_element_type=_F32)
        + jnp.dot(xb_ref[...], wb_ref[...], preferred_element_type=_F32)
        + b_ref[...]
    )


def kernel(x, edge_index, W, b):
    xs = x.reshape(N, NC, DH).transpose(1, 0, 2)
    xs = jnp.pad(xs, ((0, 0), (0, NPAD - N), (0, 0)))
    pad = jnp.full((EPAD - E,), N, jnp.int32)
    rowp = jnp.concatenate([edge_index[0], pad]).reshape(NS, NCH, C)
    colp = jnp.concatenate([edge_index[1], pad]).reshape(NS, NCH, C)

    mesh = plsc.VectorSubcoreMesh(core_axis_name="c", subcore_axis_name="s")
    sc_fn = pl.kernel(
        _sc_body,
        out_type=jax.ShapeDtypeStruct((NC, NPAD, DH), _F32),
        mesh=mesh,
        scratch_types=[
            pltpu.VMEM_SHARED((NPAD, DH), _F32),
            pltpu.VMEM_SHARED((NPAD, DH), _F32),
            pltpu.VMEM_SHARED((NPAD,), _F32),
            pltpu.VMEM((NCH, C), jnp.int32),
            pltpu.VMEM((NCH, C), jnp.int32),
            pltpu.VMEM((C, DH), _F32),
            pltpu.VMEM((RC, DH), _F32),
            pltpu.VMEM((RPT,), _F32),
            pltpu.VMEM((RPT,), _F32),
            pltpu.VMEM((RPT,), _F32),
            pltpu.VMEM((C,), _F32),
        ],
    )
    x2s = sc_fn(xs, rowp, colp)

    wt = W.T.reshape(NC, DH, D)
    blk = 1000
    out = pl.pallas_call(
        _mm_body,
        grid=(N // blk,),
        in_specs=[
            pl.BlockSpec((blk, DH), lambda i: (i, 0)),
            pl.BlockSpec((blk, DH), lambda i: (i, 0)),
            pl.BlockSpec((DH, D), lambda i: (0, 0)),
            pl.BlockSpec((DH, D), lambda i: (0, 0)),
            pl.BlockSpec((1, D), lambda i: (0, 0)),
        ],
        out_specs=pl.BlockSpec((blk, D), lambda i: (i, 0)),
        out_shape=jax.ShapeDtypeStruct((N, D), _F32),
    )(x2s[0], x2s[1], wt[0], wt[1], b.reshape(1, D))
    return out


# speedup vs baseline: 10.8116x; 10.8116x over previous
import functools

import jax
import jax.numpy as jnp
from jax import lax
from jax.experimental import pallas as pl
from jax.experimental.pallas import tpu as pltpu
from jax.experimental.pallas import tpu_sc as plsc

N = 10000
E = 320000
D = 128
DH = 64
NC = 2
NS = 16
L = 16

C = 128
G = 8
NCH = 160
NGR = NCH // G
EPT = NCH * C
EPAD = NS * EPT

RPT = 640
NPAD = NS * RPT
RC = 128
NRC = RPT // RC

_F32 = jnp.float32


def _rsqrt16(v):
    half = jnp.full((L,), 0.5, _F32)
    quart = jnp.full((L,), 0.25, _F32)
    four = jnp.full((L,), 4.0, _F32)
    three_half = jnp.full((L,), 1.5, _F32)
    c0 = jnp.full((L,), 1.1032, _F32)
    c1 = jnp.full((L,), 1.0 / 6.0, _F32)
    y = jnp.full((L,), 1.0, _F32)
    m = v
    for _ in range(10):
        big = m >= four
        y = jnp.where(big, y * half, y)
        m = jnp.where(big, m * quart, m)
    y = y * (c0 - c1 * m)
    for _ in range(4):
        y = y * (three_half - half * v * y * y)
    return y


def _scale_rows(nbuf, sref, base):
    def body(i, _):
        dv16 = sref[pl.ds(base + i, L)]
        dv = jnp.full((L,), dv16[0], _F32)
        for q in range(DH // L):
            sl = pl.ds(q * L, L)
            nbuf[i, sl] = nbuf[i, sl] * dv
        return 0
    lax.fori_loop(0, RC, body, 0)


def _fill(ref, n, val):
    vec = jnp.full((L,), val, ref.dtype)
    def body(k, _):
        ref[pl.ds(k * L, L)] = vec
        return 0
    lax.fori_loop(0, n // L, body, 0)


def _sc_body(xs, rowc, colc, out, ybuf,
             z_sh, deg_sh,
             col_v, row_v, gbuf, nbuf, degt, dtile, itile, ones_v, idxb):
    c = lax.axis_index("c")
    s = lax.axis_index("s")

    _fill(ones_v, C, 1.0)
    _fill(degt, RPT, 1.0)
    pltpu.sync_copy(degt, deg_sh.at[pl.ds(s * RPT, RPT)])
    plsc.subcore_barrier()

    def hist(g, _):
        pltpu.sync_copy(colc.at[s, pl.ds(g * G, G)], col_v)
        def hist_inner(j, _):
            pltpu.sync_copy(ones_v, deg_sh.at[col_v.at[j]], add=True)
            return 0
        lax.fori_loop(0, G, hist_inner, 0)
        return 0
    lax.fori_loop(0, NGR, hist, 0)
    plsc.subcore_barrier()

    pltpu.sync_copy(deg_sh.at[pl.ds(s * RPT, RPT)], degt)
    def dcalc(k, _):
        sl = pl.ds(k * L, L)
        y = _rsqrt16(degt[sl])
        dtile[sl] = y
        itile[sl] = y * y
        return 0
    lax.fori_loop(0, RPT // L, dcalc, 0)

    def _fill_idx(g0):
        def body(k, _):
            idxb[0, pl.ds(k * L, L)] = lax.iota(jnp.int32, L) + g0 + k * L
            return 0
        lax.fori_loop(0, RC // L, body, 0)

    def prescale(r, _):
        g0 = s * RPT + r * RC
        pltpu.sync_copy(xs.at[c, pl.ds(g0, RC), :], nbuf)
        _scale_rows(nbuf, dtile, r * RC)
        _fill_idx(g0)
        pltpu.sync_copy(nbuf, ybuf.at[c, pl.ds(g0, RC), :])
        pltpu.sync_copy(nbuf, z_sh.at[idxb.at[0]])
        return 0
    lax.fori_loop(0, NRC, prescale, 0)
    plsc.subcore_barrier()

    def edges(g, _):
        pltpu.sync_copy(colc.at[s, pl.ds(g * G, G)], col_v)
        pltpu.sync_copy(rowc.at[s, pl.ds(g * G, G)], row_v)
        def edges_inner(j, _):
            pltpu.sync_copy(ybuf.at[c].at[col_v.at[j]], gbuf)
            pltpu.sync_copy(gbuf, z_sh.at[row_v.at[j]], add=True)
            return 0
        lax.fori_loop(0, G, edges_inner, 0)
        return 0
    lax.fori_loop(0, NGR, edges, 0)
    plsc.subcore_barrier()

    def midscale(r, _):
        g0 = s * RPT + r * RC
        pltpu.sync_copy(z_sh.at[pl.ds(g0, RC)], nbuf)
        _scale_rows(nbuf, itile, r * RC)
        _fill_idx(g0)
        pltpu.sync_copy(nbuf, ybuf.at[c, pl.ds(g0, RC), :])
        pltpu.sync_copy(nbuf, z_sh.at[idxb.at[0]])
        return 0
    lax.fori_loop(0, NRC, midscale, 0)
    plsc.subcore_barrier()

    lax.fori_loop(0, NGR, edges, 0)
    plsc.subcore_barrier()

    def finalize(r, _):
        g0 = s * RPT + r * RC
        pltpu.sync_copy(z_sh.at[pl.ds(g0, RC)], nbuf)
        _scale_rows(nbuf, dtile, r * RC)
        pltpu.sync_copy(nbuf, out.at[c, pl.ds(g0, RC), :])
        return 0
    lax.fori_loop(0, NRC, finalize, 0)


def _mm_body(xa_ref, xb_ref, wa_ref, wb_ref, b_ref, o_ref):
    o_ref[...] = (
        jnp.dot(xa_ref[...], wa_ref[...], preferred_element_type=_F32)
        + jnp.dot(xb_ref[...], wb_ref[...], preferred_element_type=_F32)
        + b_ref[...]
    )


def kernel(x, edge_index, W, b):
    xs = x.reshape(N, NC, DH).transpose(1, 0, 2)
    xs = jnp.pad(xs, ((0, 0), (0, NPAD - N), (0, 0)))
    pad = jnp.full((EPAD - E,), N, jnp.int32)
    rowp = jnp.concatenate([edge_index[0], pad]).reshape(NS, NCH, C)
    colp = jnp.concatenate([edge_index[1], pad]).reshape(NS, NCH, C)

    mesh = plsc.VectorSubcoreMesh(
        core_axis_name="c", subcore_axis_name="s", num_cores=NC, num_subcores=NS
    )
    sc_fn = pl.kernel(
        _sc_body,
        out_type=(
            jax.ShapeDtypeStruct((NC, NPAD, DH), _F32),
            jax.ShapeDtypeStruct((NC, NPAD, DH), _F32),
        ),
        mesh=mesh,
        compiler_params=pltpu.CompilerParams(
            needs_layout_passes=False, use_tc_tiling_on_sc=False
        ),
        scratch_types=[
            pltpu.VMEM_SHARED((NPAD, DH), _F32),
            pltpu.VMEM_SHARED((NPAD,), _F32),
            pltpu.VMEM((G, C), jnp.int32),
            pltpu.VMEM((G, C), jnp.int32),
            pltpu.VMEM((C, DH), _F32),
            pltpu.VMEM((RC, DH), _F32),
            pltpu.VMEM((RPT,), _F32),
            pltpu.VMEM((RPT + L,), _F32),
            pltpu.VMEM((RPT + L,), _F32),
            pltpu.VMEM((C,), _F32),
            pltpu.VMEM((1, RC), jnp.int32),
        ],
    )
    x2s, _ = sc_fn(xs, rowp, colp)

    wt = W.T.reshape(NC, DH, D)
    blk = 1000
    out = pl.pallas_call(
        _mm_body,
        grid=(N // blk,),
        in_specs=[
            pl.BlockSpec((blk, DH), lambda i: (i, 0)),
            pl.BlockSpec((blk, DH), lambda i: (i, 0)),
            pl.BlockSpec((DH, D), lambda i: (0, 0)),
            pl.BlockSpec((DH, D), lambda i: (0, 0)),
            pl.BlockSpec((1, D), lambda i: (0, 0)),
        ],
        out_specs=pl.BlockSpec((blk, D), lambda i: (i, 0)),
        out_shape=jax.ShapeDtypeStruct((N, D), _F32),
    )(x2s[0], x2s[1], wt[0], wt[1], b.reshape(1, D))
    return out


# speedup vs baseline: 12.1589x; 1.1246x over previous
import functools

import jax
import jax.numpy as jnp
from jax import lax
from jax.experimental import pallas as pl
from jax.experimental.pallas import tpu as pltpu
from jax.experimental.pallas import tpu_sc as plsc

N = 10000
E = 320000
D = 128
DH = 64
NC = 2
NS = 16
L = 16

C = 128
G = 8
NCH = 160
NGR = NCH // G
EPT = NCH * C
EPAD = NS * EPT

RPT = 640
NPAD = NS * RPT
RC = 128
NRC = RPT // RC

_F32 = jnp.float32


def _rsqrt16(v):
    half = jnp.full((L,), 0.5, _F32)
    quart = jnp.full((L,), 0.25, _F32)
    four = jnp.full((L,), 4.0, _F32)
    three_half = jnp.full((L,), 1.5, _F32)
    c0 = jnp.full((L,), 1.1032, _F32)
    c1 = jnp.full((L,), 1.0 / 6.0, _F32)
    y = jnp.full((L,), 1.0, _F32)
    m = v
    for _ in range(10):
        big = m >= four
        y = jnp.where(big, y * half, y)
        m = jnp.where(big, m * quart, m)
    y = y * (c0 - c1 * m)
    for _ in range(4):
        y = y * (three_half - half * v * y * y)
    return y


def _scale_rows(nbuf, sref, base):
    def body(i, _):
        dv16 = sref[pl.ds(base + i, L)]
        dv = jnp.full((L,), dv16[0], _F32)
        for q in range(DH // L):
            sl = pl.ds(q * L, L)
            nbuf[i, sl] = nbuf[i, sl] * dv
        return 0
    lax.fori_loop(0, RC, body, 0)


def _fill(ref, n, val):
    vec = jnp.full((L,), val, ref.dtype)
    def body(k, _):
        ref[pl.ds(k * L, L)] = vec
        return 0
    lax.fori_loop(0, n // L, body, 0)


def _sc_body(xs, rowc, colc, out, ybuf,
             z_sh, deg_sh,
             col_v, row_v, gbuf, nbuf, degt, dtile, itile, ones_v, idxb,
             gs0, gs1, ss0, ss1, hsem):
    c = lax.axis_index("c")
    s = lax.axis_index("s")
    gsems = (gs0, gs1)
    ssems = (ss0, ss1)

    _fill(ones_v, C, 1.0)
    _fill(degt, RPT, 1.0)
    pltpu.sync_copy(degt, deg_sh.at[pl.ds(s * RPT, RPT)])
    plsc.subcore_barrier()

    def hist(g, _):
        pltpu.sync_copy(colc.at[s, pl.ds(g * G, G)], col_v)
        descs = [
            pltpu.async_copy(ones_v, deg_sh.at[col_v.at[j]], hsem, add=True)
            for j in range(G)
        ]
        for dsc in descs:
            dsc.wait()
        return 0
    lax.fori_loop(0, NGR, hist, 0)
    plsc.subcore_barrier()

    pltpu.sync_copy(deg_sh.at[pl.ds(s * RPT, RPT)], degt)
    def dcalc(k, _):
        sl = pl.ds(k * L, L)
        y = _rsqrt16(degt[sl])
        dtile[sl] = y
        itile[sl] = y * y
        return 0
    lax.fori_loop(0, RPT // L, dcalc, 0)

    def _fill_idx(g0):
        def body(k, _):
            idxb[0, pl.ds(k * L, L)] = lax.iota(jnp.int32, L) + g0 + k * L
            return 0
        lax.fori_loop(0, RC // L, body, 0)

    def prescale(r, _):
        g0 = s * RPT + r * RC
        pltpu.sync_copy(xs.at[c, pl.ds(g0, RC), :], nbuf)
        _scale_rows(nbuf, dtile, r * RC)
        _fill_idx(g0)
        pltpu.sync_copy(nbuf, ybuf.at[c, pl.ds(g0, RC), :])
        pltpu.sync_copy(nbuf, z_sh.at[idxb.at[0]])
        return 0
    lax.fori_loop(0, NRC, prescale, 0)
    plsc.subcore_barrier()

    def edges(g, _):
        pltpu.sync_copy(colc.at[s, pl.ds(g * G, G)], col_v)
        pltpu.sync_copy(rowc.at[s, pl.ds(g * G, G)], row_v)
        gd = [None] * G
        sd = [None] * G
        gd[0] = pltpu.async_copy(ybuf.at[c].at[col_v.at[0]], gbuf.at[0], gs0)
        for j in range(G):
            b = j & 1
            gd[j].wait()
            sd[j] = pltpu.async_copy(
                gbuf.at[b], z_sh.at[row_v.at[j]], ssems[b], add=True)
            if j + 1 < G:
                nb = (j + 1) & 1
                if j >= 1:
                    sd[j - 1].wait()
                gd[j + 1] = pltpu.async_copy(
                    ybuf.at[c].at[col_v.at[j + 1]], gbuf.at[nb], gsems[nb])
        sd[G - 2].wait()
        sd[G - 1].wait()
        return 0
    lax.fori_loop(0, NGR, edges, 0)
    plsc.subcore_barrier()

    def midscale(r, _):
        g0 = s * RPT + r * RC
        pltpu.sync_copy(z_sh.at[pl.ds(g0, RC)], nbuf)
        _scale_rows(nbuf, itile, r * RC)
        _fill_idx(g0)
        pltpu.sync_copy(nbuf, ybuf.at[c, pl.ds(g0, RC), :])
        pltpu.sync_copy(nbuf, z_sh.at[idxb.at[0]])
        return 0
    lax.fori_loop(0, NRC, midscale, 0)
    plsc.subcore_barrier()

    lax.fori_loop(0, NGR, edges, 0)
    plsc.subcore_barrier()

    def finalize(r, _):
        g0 = s * RPT + r * RC
        pltpu.sync_copy(z_sh.at[pl.ds(g0, RC)], nbuf)
        _scale_rows(nbuf, dtile, r * RC)
        pltpu.sync_copy(nbuf, out.at[c, pl.ds(g0, RC), :])
        return 0
    lax.fori_loop(0, NRC, finalize, 0)


def _mm_body(xa_ref, xb_ref, wa_ref, wb_ref, b_ref, o_ref):
    o_ref[...] = (
        jnp.dot(xa_ref[...], wa_ref[...], preferred_element_type=_F32)
        + jnp.dot(xb_ref[...], wb_ref[...], preferred_element_type=_F32)
        + b_ref[...]
    )


def kernel(x, edge_index, W, b):
    xs = x.reshape(N, NC, DH).transpose(1, 0, 2)
    xs = jnp.pad(xs, ((0, 0), (0, NPAD - N), (0, 0)))
    pad = jnp.full((EPAD - E,), N, jnp.int32)
    rowp = jnp.concatenate([edge_index[0], pad]).reshape(NS, NCH, C)
    colp = jnp.concatenate([edge_index[1], pad]).reshape(NS, NCH, C)

    mesh = plsc.VectorSubcoreMesh(
        core_axis_name="c", subcore_axis_name="s", num_cores=NC, num_subcores=NS
    )
    sc_fn = pl.kernel(
        _sc_body,
        out_type=(
            jax.ShapeDtypeStruct((NC, NPAD, DH), _F32),
            jax.ShapeDtypeStruct((NC, NPAD, DH), _F32),
        ),
        mesh=mesh,
        compiler_params=pltpu.CompilerParams(
            needs_layout_passes=False, use_tc_tiling_on_sc=False
        ),
        scratch_types=[
            pltpu.VMEM_SHARED((NPAD, DH), _F32),
            pltpu.VMEM_SHARED((NPAD,), _F32),
            pltpu.VMEM((G, C), jnp.int32),
            pltpu.VMEM((G, C), jnp.int32),
            pltpu.VMEM((2, C, DH), _F32),
            pltpu.VMEM((RC, DH), _F32),
            pltpu.VMEM((RPT,), _F32),
            pltpu.VMEM((RPT + L,), _F32),
            pltpu.VMEM((RPT + L,), _F32),
            pltpu.VMEM((C,), _F32),
            pltpu.VMEM((1, RC), jnp.int32),
            pltpu.SemaphoreType.DMA,
            pltpu.SemaphoreType.DMA,
            pltpu.SemaphoreType.DMA,
            pltpu.SemaphoreType.DMA,
            pltpu.SemaphoreType.DMA,
        ],
    )
    x2s, _ = sc_fn(xs, rowp, colp)

    wt = W.T.reshape(NC, DH, D)
    blk = 1000
    out = pl.pallas_call(
        _mm_body,
        grid=(N // blk,),
        in_specs=[
            pl.BlockSpec((blk, DH), lambda i: (i, 0)),
            pl.BlockSpec((blk, DH), lambda i: (i, 0)),
            pl.BlockSpec((DH, D), lambda i: (0, 0)),
            pl.BlockSpec((DH, D), lambda i: (0, 0)),
            pl.BlockSpec((1, D), lambda i: (0, 0)),
        ],
        out_specs=pl.BlockSpec((blk, D), lambda i: (i, 0)),
        out_shape=jax.ShapeDtypeStruct((N, D), _F32),
    )(x2s[0], x2s[1], wt[0], wt[1], b.reshape(1, D))
    return out


# speedup vs baseline: 20.5324x; 1.6887x over previous
import functools

import jax
import jax.numpy as jnp
from jax import lax
from jax.experimental import pallas as pl
from jax.experimental.pallas import tpu as pltpu
from jax.experimental.pallas import tpu_sc as plsc

N = 10000
E = 320000
D = 128
DH = 64
NC = 2
NS = 16
L = 16

C = 128
G = 8
NCH = 160
NGR = NCH // G
EPT = NCH * C
EPAD = NS * EPT

RPT = 640
NPAD = NS * RPT
RC = 128
NRC = RPT // RC

_F32 = jnp.float32


def _rsqrt16(v):
    half = jnp.full((L,), 0.5, _F32)
    quart = jnp.full((L,), 0.25, _F32)
    four = jnp.full((L,), 4.0, _F32)
    three_half = jnp.full((L,), 1.5, _F32)
    c0 = jnp.full((L,), 1.1032, _F32)
    c1 = jnp.full((L,), 1.0 / 6.0, _F32)
    y = jnp.full((L,), 1.0, _F32)
    m = v
    for _ in range(10):
        big = m >= four
        y = jnp.where(big, y * half, y)
        m = jnp.where(big, m * quart, m)
    y = y * (c0 - c1 * m)
    for _ in range(4):
        y = y * (three_half - half * v * y * y)
    return y


def _scale_rows(nbuf, sref, base):
    def body(i, _):
        dv16 = sref[pl.ds(base + i, L)]
        dv = jnp.full((L,), dv16[0], _F32)
        for q in range(DH // L):
            sl = pl.ds(q * L, L)
            nbuf[i, sl] = nbuf[i, sl] * dv
        return 0
    lax.fori_loop(0, RC, body, 0)


def _fill(ref, n, val):
    vec = jnp.full((L,), val, ref.dtype)
    def body(k, _):
        ref[pl.ds(k * L, L)] = vec
        return 0
    lax.fori_loop(0, n // L, body, 0)


def _sc_body(xs, rowc, colc, out,
             y_sh, z_sh, deg_sh,
             col_v, row_v, gbuf, nbuf, degt, dtile, itile, ones_v, idxb,
             gs0, gs1, ss0, ss1, hsem):
    c = lax.axis_index("c")
    s = lax.axis_index("s")
    gsems = (gs0, gs1)
    ssems = (ss0, ss1)

    _fill(ones_v, C, 1.0)
    _fill(degt, RPT, 1.0)
    pltpu.sync_copy(degt, deg_sh.at[pl.ds(s * RPT, RPT)])
    plsc.subcore_barrier()

    def hist(g, _):
        pltpu.sync_copy(colc.at[s, pl.ds(g * G, G)], col_v)
        descs = [
            pltpu.async_copy(ones_v, deg_sh.at[col_v.at[j]], hsem, add=True)
            for j in range(G)
        ]
        for dsc in descs:
            dsc.wait()
        return 0
    lax.fori_loop(0, NGR, hist, 0)
    plsc.subcore_barrier()

    pltpu.sync_copy(deg_sh.at[pl.ds(s * RPT, RPT)], degt)
    def dcalc(k, _):
        sl = pl.ds(k * L, L)
        y = _rsqrt16(degt[sl])
        dtile[sl] = y
        itile[sl] = y * y
        return 0
    lax.fori_loop(0, RPT // L, dcalc, 0)

    def _fill_idx(g0):
        def body(k, _):
            idxb[0, pl.ds(k * L, L)] = lax.iota(jnp.int32, L) + g0 + k * L
            return 0
        lax.fori_loop(0, RC // L, body, 0)

    def prescale(r, _):
        g0 = s * RPT + r * RC
        pltpu.sync_copy(xs.at[c, pl.ds(g0, RC), :], nbuf)
        _scale_rows(nbuf, dtile, r * RC)
        _fill_idx(g0)
        pltpu.sync_copy(nbuf, y_sh.at[idxb.at[0]])
        pltpu.sync_copy(nbuf, z_sh.at[idxb.at[0]])
        return 0
    lax.fori_loop(0, NRC, prescale, 0)
    plsc.subcore_barrier()

    def edges(g, _):
        pltpu.sync_copy(colc.at[s, pl.ds(g * G, G)], col_v)
        pltpu.sync_copy(rowc.at[s, pl.ds(g * G, G)], row_v)
        gd = [None] * G
        sd = [None] * G
        gd[0] = pltpu.async_copy(y_sh.at[col_v.at[0]], gbuf.at[0], gs0)
        for j in range(G):
            b = j & 1
            gd[j].wait()
            sd[j] = pltpu.async_copy(
                gbuf.at[b], z_sh.at[row_v.at[j]], ssems[b], add=True)
            if j + 1 < G:
                nb = (j + 1) & 1
                if j >= 1:
                    sd[j - 1].wait()
                gd[j + 1] = pltpu.async_copy(
                    y_sh.at[col_v.at[j + 1]], gbuf.at[nb], gsems[nb])
        sd[G - 2].wait()
        sd[G - 1].wait()
        return 0
    lax.fori_loop(0, NGR, edges, 0)
    plsc.subcore_barrier()

    def midscale(r, _):
        g0 = s * RPT + r * RC
        pltpu.sync_copy(z_sh.at[pl.ds(g0, RC)], nbuf)
        _scale_rows(nbuf, itile, r * RC)
        _fill_idx(g0)
        pltpu.sync_copy(nbuf, y_sh.at[idxb.at[0]])
        pltpu.sync_copy(nbuf, z_sh.at[idxb.at[0]])
        return 0
    lax.fori_loop(0, NRC, midscale, 0)
    plsc.subcore_barrier()

    lax.fori_loop(0, NGR, edges, 0)
    plsc.subcore_barrier()

    def finalize(r, _):
        g0 = s * RPT + r * RC
        pltpu.sync_copy(z_sh.at[pl.ds(g0, RC)], nbuf)
        _scale_rows(nbuf, dtile, r * RC)
        pltpu.sync_copy(nbuf, out.at[c, pl.ds(g0, RC), :])
        return 0
    lax.fori_loop(0, NRC, finalize, 0)


def _mm_body(xa_ref, xb_ref, wa_ref, wb_ref, b_ref, o_ref):
    o_ref[...] = (
        jnp.dot(xa_ref[...], wa_ref[...], preferred_element_type=_F32)
        + jnp.dot(xb_ref[...], wb_ref[...], preferred_element_type=_F32)
        + b_ref[...]
    )


def kernel(x, edge_index, W, b):
    xs = x.reshape(N, NC, DH).transpose(1, 0, 2)
    xs = jnp.pad(xs, ((0, 0), (0, NPAD - N), (0, 0)))
    pad = jnp.full((EPAD - E,), N, jnp.int32)
    rowp = jnp.concatenate([edge_index[0], pad]).reshape(NS, NCH, C)
    colp = jnp.concatenate([edge_index[1], pad]).reshape(NS, NCH, C)

    mesh = plsc.VectorSubcoreMesh(
        core_axis_name="c", subcore_axis_name="s", num_cores=NC, num_subcores=NS
    )
    sc_fn = pl.kernel(
        _sc_body,
        out_type=jax.ShapeDtypeStruct((NC, NPAD, DH), _F32),
        mesh=mesh,
        compiler_params=pltpu.CompilerParams(
            needs_layout_passes=False, use_tc_tiling_on_sc=False
        ),
        scratch_types=[
            pltpu.VMEM_SHARED((NPAD, DH), _F32),
            pltpu.VMEM_SHARED((NPAD, DH), _F32),
            pltpu.VMEM_SHARED((NPAD,), _F32),
            pltpu.VMEM((G, C), jnp.int32),
            pltpu.VMEM((G, C), jnp.int32),
            pltpu.VMEM((2, C, DH), _F32),
            pltpu.VMEM((RC, DH), _F32),
            pltpu.VMEM((RPT,), _F32),
            pltpu.VMEM((RPT + L,), _F32),
            pltpu.VMEM((RPT + L,), _F32),
            pltpu.VMEM((C,), _F32),
            pltpu.VMEM((1, RC), jnp.int32),
            pltpu.SemaphoreType.DMA,
            pltpu.SemaphoreType.DMA,
            pltpu.SemaphoreType.DMA,
            pltpu.SemaphoreType.DMA,
            pltpu.SemaphoreType.DMA,
        ],
    )
    x2s = sc_fn(xs, rowp, colp)

    wt = W.T.reshape(NC, DH, D)
    blk = 1000
    out = pl.pallas_call(
        _mm_body,
        grid=(N // blk,),
        in_specs=[
            pl.BlockSpec((blk, DH), lambda i: (i, 0)),
            pl.BlockSpec((blk, DH), lambda i: (i, 0)),
            pl.BlockSpec((DH, D), lambda i: (0, 0)),
            pl.BlockSpec((DH, D), lambda i: (0, 0)),
            pl.BlockSpec((1, D), lambda i: (0, 0)),
        ],
        out_specs=pl.BlockSpec((blk, D), lambda i: (i, 0)),
        out_shape=jax.ShapeDtypeStruct((N, D), _F32),
    )(x2s[0], x2s[1], wt[0], wt[1], b.reshape(1, D))
    return out


# speedup vs baseline: 22.8630x; 1.1135x over previous
import functools

import jax
import jax.numpy as jnp
from jax import lax
from jax.experimental import pallas as pl
from jax.experimental.pallas import tpu as pltpu
from jax.experimental.pallas import tpu_sc as plsc

N = 10000
E = 320000
D = 128
DH = 64
NC = 2
NS = 16
L = 16

C = 128
G = 8
NCH = 160
NGR = NCH // G
EPT = NCH * C
EPAD = NS * EPT

RPT = 640
NPAD = NS * RPT
RC = 128
NRC = RPT // RC

_F32 = jnp.float32


def _rsqrt16(v):
    half = jnp.full((L,), 0.5, _F32)
    quart = jnp.full((L,), 0.25, _F32)
    four = jnp.full((L,), 4.0, _F32)
    three_half = jnp.full((L,), 1.5, _F32)
    c0 = jnp.full((L,), 1.1032, _F32)
    c1 = jnp.full((L,), 1.0 / 6.0, _F32)
    y = jnp.full((L,), 1.0, _F32)
    m = v
    for _ in range(10):
        big = m >= four
        y = jnp.where(big, y * half, y)
        m = jnp.where(big, m * quart, m)
    y = y * (c0 - c1 * m)
    for _ in range(4):
        y = y * (three_half - half * v * y * y)
    return y


def _scale_rows(nbuf, sref, base):
    def body(i, _):
        dv16 = sref[pl.ds(base + i, L)]
        dv = jnp.full((L,), dv16[0], _F32)
        for q in range(DH // L):
            sl = pl.ds(q * L, L)
            nbuf[i, sl] = nbuf[i, sl] * dv
        return 0
    lax.fori_loop(0, RC, body, 0)


def _fill(ref, n, val):
    vec = jnp.full((L,), val, ref.dtype)
    def body(k, _):
        ref[pl.ds(k * L, L)] = vec
        return 0
    lax.fori_loop(0, n // L, body, 0)


def _sc_body(xs, rowc, colc, out,
             y_sh, z_sh, deg_sh,
             col_v, row_v, gbuf, nbuf, degt, dtile, itile, ones_v, idxb,
             gs0, gs1, ss0, ss1, hsem, isem):
    c = lax.axis_index("c")
    s = lax.axis_index("s")
    gsems = (gs0, gs1)
    ssems = (ss0, ss1)

    _fill(ones_v, C, 1.0)
    _fill(degt, RPT, 1.0)
    pltpu.sync_copy(degt, deg_sh.at[pl.ds(s * RPT, RPT)])
    plsc.subcore_barrier()

    def _load_idx(g, slot, ref, src):
        return pltpu.async_copy(src.at[s, pl.ds(g * G, G)], ref.at[slot], isem)

    def _hist_group(slot):
        descs = [
            pltpu.async_copy(
                ones_v, deg_sh.at[col_v.at[slot, j]], hsem, add=True)
            for j in range(G)
        ]
        for dsc in descs:
            dsc.wait()

    _load_idx(0, 0, col_v, colc)
    def hist(h, _):
        g0 = 2 * h
        pltpu.make_async_copy(
            colc.at[s, pl.ds(g0 * G, G)], col_v.at[0], isem).wait()
        _load_idx(g0 + 1, 1, col_v, colc)
        _hist_group(0)
        pltpu.make_async_copy(
            colc.at[s, pl.ds(g0 * G, G)], col_v.at[1], isem).wait()
        @pl.when(h + 1 < NGR // 2)
        def _():
            _load_idx(g0 + 2, 0, col_v, colc)
        _hist_group(1)
        return 0
    lax.fori_loop(0, NGR // 2, hist, 0)
    plsc.subcore_barrier()

    pltpu.sync_copy(deg_sh.at[pl.ds(s * RPT, RPT)], degt)
    def dcalc(k, _):
        sl = pl.ds(k * L, L)
        y = _rsqrt16(degt[sl])
        dtile[sl] = y
        itile[sl] = y * y
        return 0
    lax.fori_loop(0, RPT // L, dcalc, 0)

    def _fill_idx(g0):
        def body(k, _):
            idxb[0, pl.ds(k * L, L)] = lax.iota(jnp.int32, L) + g0 + k * L
            return 0
        lax.fori_loop(0, RC // L, body, 0)

    def prescale(r, _):
        g0 = s * RPT + r * RC
        pltpu.sync_copy(xs.at[c, pl.ds(g0, RC), :], nbuf)
        _scale_rows(nbuf, dtile, r * RC)
        _fill_idx(g0)
        pltpu.sync_copy(nbuf, y_sh.at[idxb.at[0]])
        pltpu.sync_copy(nbuf, z_sh.at[idxb.at[0]])
        return 0
    lax.fori_loop(0, NRC, prescale, 0)
    plsc.subcore_barrier()

    def _edge_group(slot):
        gd = [None] * G
        sd = [None] * G
        gd[0] = pltpu.async_copy(y_sh.at[col_v.at[slot, 0]], gbuf.at[0], gs0)
        for j in range(G):
            b = j & 1
            gd[j].wait()
            sd[j] = pltpu.async_copy(
                gbuf.at[b], z_sh.at[row_v.at[slot, j]], ssems[b], add=True)
            if j + 1 < G:
                nb = (j + 1) & 1
                if j >= 1:
                    sd[j - 1].wait()
                gd[j + 1] = pltpu.async_copy(
                    y_sh.at[col_v.at[slot, j + 1]], gbuf.at[nb], gsems[nb])
        sd[G - 2].wait()
        sd[G - 1].wait()

    def _wait_idx(ref, slot):
        pltpu.make_async_copy(
            colc.at[s, pl.ds(0, G)], ref.at[slot], isem).wait()

    def _prefetch(g, slot):
        pltpu.async_copy(colc.at[s, pl.ds(g * G, G)], col_v.at[slot], isem)
        pltpu.async_copy(rowc.at[s, pl.ds(g * G, G)], row_v.at[slot], isem)

    def edges(h, _):
        g0 = 2 * h
        _wait_idx(col_v, 0)
        _wait_idx(row_v, 0)
        _prefetch(g0 + 1, 1)
        _edge_group(0)
        _wait_idx(col_v, 1)
        _wait_idx(row_v, 1)
        @pl.when(h + 1 < NGR // 2)
        def _():
            _prefetch(g0 + 2, 0)
        _edge_group(1)
        return 0

    _prefetch(0, 0)
    lax.fori_loop(0, NGR // 2, edges, 0)
    plsc.subcore_barrier()

    def midscale(r, _):
        g0 = s * RPT + r * RC
        pltpu.sync_copy(z_sh.at[pl.ds(g0, RC)], nbuf)
        _scale_rows(nbuf, itile, r * RC)
        _fill_idx(g0)
        pltpu.sync_copy(nbuf, y_sh.at[idxb.at[0]])
        pltpu.sync_copy(nbuf, z_sh.at[idxb.at[0]])
        return 0
    lax.fori_loop(0, NRC, midscale, 0)
    plsc.subcore_barrier()

    _prefetch(0, 0)
    lax.fori_loop(0, NGR // 2, edges, 0)
    plsc.subcore_barrier()

    def finalize(r, _):
        g0 = s * RPT + r * RC
        pltpu.sync_copy(z_sh.at[pl.ds(g0, RC)], nbuf)
        _scale_rows(nbuf, dtile, r * RC)
        pltpu.sync_copy(nbuf, out.at[c, pl.ds(g0, RC), :])
        return 0
    lax.fori_loop(0, NRC, finalize, 0)


def _mm_body(xa_ref, xb_ref, wa_ref, wb_ref, b_ref, o_ref):
    o_ref[...] = (
        jnp.dot(xa_ref[...], wa_ref[...], preferred_element_type=_F32)
        + jnp.dot(xb_ref[...], wb_ref[...], preferred_element_type=_F32)
        + b_ref[...]
    )


def kernel(x, edge_index, W, b):
    xs = x.reshape(N, NC, DH).transpose(1, 0, 2)
    xs = jnp.pad(xs, ((0, 0), (0, NPAD - N), (0, 0)))
    pad = jnp.full((EPAD - E,), N, jnp.int32)
    rowp = jnp.concatenate([edge_index[0], pad]).reshape(NS, NCH, C)
    colp = jnp.concatenate([edge_index[1], pad]).reshape(NS, NCH, C)

    mesh = plsc.VectorSubcoreMesh(
        core_axis_name="c", subcore_axis_name="s", num_cores=NC, num_subcores=NS
    )
    sc_fn = pl.kernel(
        _sc_body,
        out_type=jax.ShapeDtypeStruct((NC, NPAD, DH), _F32),
        mesh=mesh,
        compiler_params=pltpu.CompilerParams(
            needs_layout_passes=False, use_tc_tiling_on_sc=False
        ),
        scratch_types=[
            pltpu.VMEM_SHARED((NPAD, DH), _F32),
            pltpu.VMEM_SHARED((NPAD, DH), _F32),
            pltpu.VMEM_SHARED((NPAD,), _F32),
            pltpu.VMEM((2, G, C), jnp.int32),
            pltpu.VMEM((2, G, C), jnp.int32),
            pltpu.VMEM((2, C, DH), _F32),
            pltpu.VMEM((RC, DH), _F32),
            pltpu.VMEM((RPT,), _F32),
            pltpu.VMEM((RPT + L,), _F32),
            pltpu.VMEM((RPT + L,), _F32),
            pltpu.VMEM((C,), _F32),
            pltpu.VMEM((1, RC), jnp.int32),
            pltpu.SemaphoreType.DMA,
            pltpu.SemaphoreType.DMA,
            pltpu.SemaphoreType.DMA,
            pltpu.SemaphoreType.DMA,
            pltpu.SemaphoreType.DMA,
            pltpu.SemaphoreType.DMA,
        ],
    )
    x2s = sc_fn(xs, rowp, colp)

    wt = W.T.reshape(NC, DH, D)
    blk = 1000
    out = pl.pallas_call(
        _mm_body,
        grid=(N // blk,),
        in_specs=[
            pl.BlockSpec((blk, DH), lambda i: (i, 0)),
            pl.BlockSpec((blk, DH), lambda i: (i, 0)),
            pl.BlockSpec((DH, D), lambda i: (0, 0)),
            pl.BlockSpec((DH, D), lambda i: (0, 0)),
            pl.BlockSpec((1, D), lambda i: (0, 0)),
        ],
        out_specs=pl.BlockSpec((blk, D), lambda i: (i, 0)),
        out_shape=jax.ShapeDtypeStruct((N, D), _F32),
    )(x2s[0], x2s[1], wt[0], wt[1], b.reshape(1, D))
    return out
